# baseline (speedup 1.0000x reference)
import jax
import jax.numpy as jnp
from jax.experimental import pallas as pl


def _fps(xyz, npoint):
    def one(pts):
        n = pts.shape[0]
        def body(i, carry):
            dist, farthest, cent = carry
            cent = cent.at[i].set(farthest)
            c = pts[farthest]
            d = jnp.sum((pts - c) ** 2, axis=-1)
            dist = jnp.minimum(dist, d)
            farthest = jnp.argmax(dist).astype(jnp.int32)
            return dist, farthest, cent
        dist0 = jnp.full((n,), 1e10, dtype=pts.dtype)
        cent0 = jnp.zeros((npoint,), dtype=jnp.int32)
        _, _, cent = jax.lax.fori_loop(0, npoint, body, (dist0, jnp.int32(0), cent0))
        return cent
    return jax.vmap(one)(xyz)


def _knn(new_xyz, xyz, k):
    d = jnp.sum((new_xyz[:, :, None, :] - xyz[:, None, :, :]) ** 2, axis=-1)
    _, idx = jax.lax.top_k(-d, k)
    return idx


def _set_abstraction(xyz, feat, W, b, npoint, nsample, group_all):
    feat_t = jnp.transpose(feat, (0, 2, 1))
    if group_all:
        grouped = feat_t[:, None, :, :]
        new_xyz = jnp.zeros((xyz.shape[0], 1, 3), dtype=xyz.dtype)
    else:
        fps_idx = _fps(jax.lax.stop_gradient(xyz), npoint)
        new_xyz = jnp.take_along_axis(xyz, fps_idx[:, :, None], axis=1)
        idx = _knn(jax.lax.stop_gradient(new_xyz), jax.lax.stop_gradient(xyz), nsample)
        grouped = jnp.take_along_axis(feat_t[:, None, :, :], idx[:, :, :, None], axis=2)
    h = jax.nn.relu(jnp.einsum('bskc,oc->bsko', grouped, W) + b)
    new_feat = jnp.max(h, axis=2)
    return new_xyz, jnp.transpose(new_feat, (0, 2, 1))


def _stn(xyz, s1W, s1b, s2W, s2b, s3W, s3b, f1W, f1b, f2W, f2b, f3W, f3b):
    h = jax.nn.relu(xyz @ s1W.T + s1b)
    h = jax.nn.relu(h @ s2W.T + s2b)
    h = jax.nn.relu(h @ s3W.T + s3b)
    g = jnp.max(h, axis=1)
    g = jax.nn.relu(g @ f1W.T + f1b)
    g = jax.nn.relu(g @ f2W.T + f2b)
    t = g @ f3W.T + f3b
    iden = jnp.eye(3, dtype=xyz.dtype).reshape(9)
    return (t + iden).reshape(-1, 3, 3)


def _final_linear_kernel(flat_ref, w_ref, b_ref, out_ref):
    out_ref[...] = (
        jnp.dot(flat_ref[...], w_ref[...], preferred_element_type=jnp.float32)
        + b_ref[...]
    )


def kernel(xyz, s1W, s1b, s2W, s2b, s3W, s3b, f1W, f1b, f2W, f2b, f3W, f3b, saW1, sab1, saW2, sab2, saW3, sab3, finW, finb):
    trans = _stn(xyz, s1W, s1b, s2W, s2b, s3W, s3b, f1W, f1b, f2W, f2b, f3W, f3b)
    xyz2 = jnp.einsum('bnd,bde->bne', xyz, trans)
    feat0 = jnp.transpose(xyz2, (0, 2, 1))
    l1_xyz, l1_feat = _set_abstraction(xyz2, feat0, saW1, sab1, 512, 32, False)
    l2_xyz, l2_feat = _set_abstraction(l1_xyz, l1_feat, saW2, sab2, 128, 64, False)
    _, l3_feat = _set_abstraction(l2_xyz, l2_feat, saW3, sab3, None, None, True)
    flat = l3_feat.reshape(xyz.shape[0], -1)
    out = pl.pallas_call(
        _final_linear_kernel,
        out_shape=jax.ShapeDtypeStruct((flat.shape[0], finW.shape[0]), jnp.float32),
    )(flat, finW.T, finb[None, :])
    return out


# speedup vs baseline: 9.1172x; 9.1172x over previous
import functools

import jax
import jax.numpy as jnp
from jax.experimental import pallas as pl

_BIG = 1e30


def _row(a, d):
    return a[d : d + 1, :]


def _scalar11(vec_1x, k):
    iota = jax.lax.broadcasted_iota(jnp.int32, vec_1x.shape, 1)
    return jnp.sum(jnp.where(iota == k, vec_1x, 0.0), axis=1, keepdims=True)


def _stn_kernel(x_ref, s1W, s1b, s2W, s2b, s3W, s3b, f1W, f1b, f2W, f2b,
                f3W, f3b, saW1, sab1, xTout, h1out):
    x = x_ref[0]
    h = jnp.maximum(jnp.dot(x, s1W[...], preferred_element_type=jnp.float32) + s1b[...], 0.0)
    h = jnp.maximum(jnp.dot(h, s2W[...], preferred_element_type=jnp.float32) + s2b[...], 0.0)
    h = jnp.maximum(jnp.dot(h, s3W[...], preferred_element_type=jnp.float32) + s3b[...], 0.0)
    g = jnp.max(h, axis=0, keepdims=True)
    g = jnp.maximum(jnp.dot(g, f1W[...], preferred_element_type=jnp.float32) + f1b[...], 0.0)
    g = jnp.maximum(jnp.dot(g, f2W[...], preferred_element_type=jnp.float32) + f2b[...], 0.0)
    t9 = jnp.dot(g, f3W[...], preferred_element_type=jnp.float32) + f3b[...]

    tr = jnp.concatenate([t9[:, 0:3], t9[:, 3:6], t9[:, 6:9]], axis=0)
    ei = jax.lax.broadcasted_iota(jnp.int32, (3, 3), 0)
    ej = jax.lax.broadcasted_iota(jnp.int32, (3, 3), 1)
    tr = tr + jnp.where(ei == ej, 1.0, 0.0)
    xyz2 = jnp.dot(x, tr, preferred_element_type=jnp.float32)
    rows = [jnp.transpose(xyz2[:, e : e + 1]) for e in range(3)]
    xTout[0] = jnp.concatenate(rows, axis=0)
    h1out[0] = jnp.maximum(
        jnp.dot(xyz2, saW1[...], preferred_element_type=jnp.float32) + sab1[...], 0.0
    )


def _fps_kernel(xT_ref, q_ref, *, npoint, n):
    x = xT_ref[:, 0, :]
    y = xT_ref[:, 1, :]
    z = xT_ref[:, 2, :]
    bsz = x.shape[0]
    iota_n = jax.lax.broadcasted_iota(jnp.int32, (bsz, n), 1)
    iota_s = jax.lax.broadcasted_iota(jnp.int32, (bsz, npoint), 1)

    def body(i, carry):
        dist, far, qx, qy, qz = carry
        sel = iota_n == far
        cx = jnp.sum(jnp.where(sel, x, 0.0), axis=1, keepdims=True)
        cy = jnp.sum(jnp.where(sel, y, 0.0), axis=1, keepdims=True)
        cz = jnp.sum(jnp.where(sel, z, 0.0), axis=1, keepdims=True)
        wr = iota_s == i
        qx = jnp.where(wr, cx, qx)
        qy = jnp.where(wr, cy, qy)
        qz = jnp.where(wr, cz, qz)
        dx = x - cx
        dy = y - cy
        dz = z - cz
        d = dx * dx + dy * dy + dz * dz
        dist = jnp.minimum(dist, d)
        mx = jnp.max(dist, axis=1, keepdims=True)
        far = jnp.min(
            jnp.where(dist == mx, iota_n, 2 * n), axis=1, keepdims=True
        )
        return dist, far, qx, qy, qz

    dist0 = jnp.full((bsz, n), 1e10, dtype=jnp.float32)
    far0 = jnp.zeros((bsz, 1), jnp.int32)
    q0 = jnp.zeros((bsz, npoint), jnp.float32)
    _, _, qx, qy, qz = jax.lax.fori_loop(0, npoint, body, (dist0, far0, q0, q0, q0))
    q_ref[:, 0, :] = qx
    q_ref[:, 1, :] = qy
    q_ref[:, 2, :] = qz


def _knn_pool(pT, qT, H, k, s, n):
    qc = [jnp.transpose(qT[d : d + 1, :]) for d in range(3)]
    dx = qc[0] - _row(pT, 0)
    dy = qc[1] - _row(pT, 1)
    dz = qc[2] - _row(pT, 2)
    D = dx * dx + dy * dy + dz * dz
    iota = jax.lax.broadcasted_iota(jnp.int32, (s, n), 1)
    acc0 = jnp.zeros((s, H.shape[1]), jnp.float32)

    def body(_, carry):
        D, acc = carry
        m = jnp.min(D, axis=1, keepdims=True)
        jm = jnp.min(jnp.where(D == m, iota, 2 * n), axis=1, keepdims=True)
        ohb = iota == jm
        oh = jnp.where(ohb, 1.0, 0.0)
        g = jnp.dot(oh, H, preferred_element_type=jnp.float32)
        acc = jnp.maximum(acc, g)
        D = jnp.where(ohb, _BIG, D)
        return D, acc

    _, acc = jax.lax.fori_loop(0, k, body, (D, acc0))
    return acc


def _sa1_kernel(xT_ref, qT_ref, h1_ref, saW2, sab2, h2out, *, k, s, n):
    l1f = _knn_pool(xT_ref[0], qT_ref[0], h1_ref[0], k, s, n)
    h2out[0] = jnp.maximum(
        jnp.dot(l1f, saW2[...], preferred_element_type=jnp.float32) + sab2[...], 0.0
    )


def _sa2_kernel(pT_ref, qT_ref, h2_ref, saW3, sab3, finW, finb, out_ref, *, k, s, n):
    l2f = _knn_pool(pT_ref[0], qT_ref[0], h2_ref[0], k, s, n)
    h3 = jnp.maximum(
        jnp.dot(l2f, saW3[...], preferred_element_type=jnp.float32) + sab3[...], 0.0
    )
    g = jnp.max(h3, axis=0, keepdims=True)
    out_ref[0] = jnp.dot(g, finW[...], preferred_element_type=jnp.float32) + finb[...]


def _full(shape):
    nd = len(shape)
    return pl.BlockSpec(shape, lambda *b: (0,) * nd)


def _batched(shape):
    nd = len(shape)
    return pl.BlockSpec((1,) + shape[1:], lambda b: (b,) + (0,) * (nd - 1))


def kernel(xyz, s1W, s1b, s2W, s2b, s3W, s3b, f1W, f1b, f2W, f2b, f3W, f3b,
           saW1, sab1, saW2, sab2, saW3, sab3, finW, finb):
    B, N, _ = xyz.shape
    f32 = jnp.float32

    wts = [s1W.T, s1b[None], s2W.T, s2b[None], s3W.T, s3b[None],
           f1W.T, f1b[None], f2W.T, f2b[None], f3W.T, f3b[None],
           saW1.T, sab1[None]]

    xT, H1 = pl.pallas_call(
            _stn_kernel,
            grid=(B,),
            in_specs=[_batched((1, N, 3))] + [_full(w.shape) for w in wts],
            out_specs=[_batched((1, 3, N)), _batched((1, N, 128))],
            out_shape=[jax.ShapeDtypeStruct((B, 3, N), f32),
                       jax.ShapeDtypeStruct((B, N, 128), f32)],
        )(xyz, *wts)

    q1T = pl.pallas_call(
        functools.partial(_fps_kernel, npoint=512, n=N),
        in_specs=[_full((B, 3, N))],
        out_specs=_full((B, 3, 512)),
        out_shape=jax.ShapeDtypeStruct((B, 3, 512), f32),
    )(xT)

    H2 = pl.pallas_call(
        functools.partial(_sa1_kernel, k=32, s=512, n=N),
        grid=(B,),
        in_specs=[_batched((1, 3, N)), _batched((1, 3, 512)),
                  _batched((1, N, 128)), _full((128, 256)), _full((1, 256))],
        out_specs=_batched((1, 512, 256)),
        out_shape=jax.ShapeDtypeStruct((B, 512, 256), f32),
    )(xT, q1T, H1, saW2.T, sab2[None])

    q2T = pl.pallas_call(
        functools.partial(_fps_kernel, npoint=128, n=512),
        in_specs=[_full((B, 3, 512))],
        out_specs=_full((B, 3, 128)),
        out_shape=jax.ShapeDtypeStruct((B, 3, 128), f32),
    )(q1T)

    out = pl.pallas_call(
        functools.partial(_sa2_kernel, k=64, s=128, n=512),
        grid=(B,),
        in_specs=[_batched((1, 3, 512)), _batched((1, 3, 128)),
                  _batched((1, 512, 256)), _full((256, 1024)), _full((1, 1024)),
                  _full((1024, 256)), _full((1, 256))],
        out_specs=_batched((1, 1, 256)),
        out_shape=jax.ShapeDtypeStruct((B, 1, 256), f32),
    )(q1T, q2T, H2, saW3.T, sab3[None], finW.T, finb[None])

    return out.reshape(B, 256)
